# baseline (speedup 1.0000x reference)
import functools
import jax
import jax.numpy as jnp
from jax import lax
from jax.experimental import pallas as pl
from jax.experimental.pallas import tpu as pltpu
from jax.experimental.pallas import tpu_sc as plsc

F32 = jnp.float32
I32 = jnp.int32
RAW = 40962
NEW = 163842
OUT_CH = 32
TBL = RAW * 7

NW = 32
NEWP = 165888

TOP_CHUNKS_PW = 12
TOP_PER_W = TOP_CHUNKS_PW * 128
TOP_PAD = NW * TOP_PER_W
DOWN_N = (NEW - RAW) * 2
DOWN_CHUNKS_PW = 60
DOWN_PER_W = DOWN_CHUNKS_PW * 128

GS_CHUNKS_PW = 322
DEST_PER_W = GS_CHUNKS_PW * 16
V_PAD = NW * DEST_PER_W
IDX_PER_W = DEST_PER_W * 7

_sc_mesh = plsc.VectorSubcoreMesh(
    core_axis_name="c", subcore_axis_name="s", num_cores=2, num_subcores=16)


@functools.partial(
    pl.kernel,
    out_type=jax.ShapeDtypeStruct((NEWP, 32), F32),
    mesh=_sc_mesh,
    compiler_params=pltpu.CompilerParams(use_tc_tiling_on_sc=False),
    scratch_types=[pltpu.VMEM((TOP_PER_W,), I32),
                   pltpu.VMEM((DOWN_PER_W,), I32),
                   pltpu.VMEM((128, 32), F32),
                   pltpu.VMEM((128, 32), F32),
                   pltpu.VMEM((128, 16), F32),
                   pltpu.VMEM((128, 16), F32),
                   pltpu.VMEM((64, 32), F32),
                   pltpu.SemaphoreType.DMA,
                   pltpu.SemaphoreType.DMA],
)
def _upconv_gather(h32, hpa, ti, di, out,
                   tiv, div, ta, tb, da, db, bridge, semA, semB):
    wid = lax.axis_index("s") * jnp.int32(2) + lax.axis_index("c")

    tbase = wid * jnp.int32(TOP_PER_W)
    pltpu.sync_copy(ti.at[pl.ds(tbase, TOP_PER_W)], tiv)

    def t_start(j, buf, sem):
        pltpu.async_copy(h32.at[tiv.at[pl.ds(j * jnp.int32(128), 128)]],
                         buf, sem)

    def t_write(buf, j):
        off = tbase + j * jnp.int32(128)
        full = off + jnp.int32(128) <= jnp.int32(RAW)
        part = jnp.logical_and(jnp.logical_not(full), off < jnp.int32(RAW))

        @pl.when(full)
        def _():
            pltpu.sync_copy(buf, out.at[pl.ds(off, 128)])

        @pl.when(part)
        def _():
            pltpu.sync_copy(buf.at[pl.ds(jnp.int32(0), 2)],
                            out.at[pl.ds(off, 2)])

    t_start(jnp.int32(0), ta, semA)

    def t_body(jj, carry):
        j0 = jj * jnp.int32(2)
        j1 = j0 + jnp.int32(1)
        t_start(j1, tb, semB)
        pltpu.make_async_copy(h32.at[pl.ds(jnp.int32(0), 128)], ta, semA).wait()
        t_write(ta, j0)

        @pl.when(jj < jnp.int32(TOP_CHUNKS_PW // 2 - 1))
        def _():
            t_start(j0 + jnp.int32(2), ta, semA)

        pltpu.make_async_copy(h32.at[pl.ds(jnp.int32(0), 128)], tb, semB).wait()
        t_write(tb, j1)
        return carry
    lax.fori_loop(jnp.int32(0), jnp.int32(TOP_CHUNKS_PW // 2), t_body,
                  jnp.int32(0))

    dbase = wid * jnp.int32(DOWN_PER_W)
    obase = jnp.int32(RAW) + wid * jnp.int32(DOWN_PER_W // 2)
    pltpu.sync_copy(di.at[pl.ds(dbase, DOWN_PER_W)], div)

    def d_start(j, buf, sem):
        pltpu.async_copy(hpa.at[div.at[pl.ds(j * jnp.int32(128), 128)]],
                         buf, sem)

    def d_write(buf, j):
        for r in range(64):
            bridge[r, pl.ds(0, 16)] = buf[2 * r, pl.ds(0, 16)]
            bridge[r, pl.ds(16, 16)] = buf[2 * r + 1, pl.ds(0, 16)]
        pltpu.sync_copy(bridge, out.at[pl.ds(obase + j * jnp.int32(64), 64)])

    d_start(jnp.int32(0), da, semA)

    def d_body(jj, carry):
        j0 = jj * jnp.int32(2)
        j1 = j0 + jnp.int32(1)
        d_start(j1, db, semB)
        pltpu.make_async_copy(hpa.at[pl.ds(jnp.int32(0), 128)], da, semA).wait()
        d_write(da, j0)

        @pl.when(jj < jnp.int32(DOWN_CHUNKS_PW // 2 - 1))
        def _():
            d_start(j0 + jnp.int32(2), da, semA)

        pltpu.make_async_copy(hpa.at[pl.ds(jnp.int32(0), 128)], db, semB).wait()
        d_write(db, j1)
        return carry
    lax.fori_loop(jnp.int32(0), jnp.int32(DOWN_CHUNKS_PW // 2), d_body,
                  jnp.int32(0))


@functools.partial(
    pl.kernel,
    out_type=jax.ShapeDtypeStruct((V_PAD, 32), F32),
    mesh=_sc_mesh,
    compiler_params=pltpu.CompilerParams(use_tc_tiling_on_sc=False),
    scratch_types=[pltpu.VMEM((IDX_PER_W,), I32),
                   pltpu.VMEM((112,), I32),
                   pltpu.VMEM((112,), I32),
                   pltpu.VMEM((112, 32), F32),
                   pltpu.VMEM((112, 32), F32),
                   pltpu.VMEM((16, 32), F32),
                   pltpu.SemaphoreType.DMA,
                   pltpu.SemaphoreType.DMA],
)
def _gather_sum7(pf, nidx, out, idxv, ia, ib, ra, rb, acc, semA, semB):
    wid = lax.axis_index("s") * jnp.int32(2) + lax.axis_index("c")
    ibase = wid * jnp.int32(IDX_PER_W)
    obase = wid * jnp.int32(DEST_PER_W)
    pltpu.sync_copy(nidx.at[pl.ds(ibase, IDX_PER_W)], idxv)

    pats = [lax.rem(lax.iota(I32, 16) + jnp.int32(16 * g), jnp.int32(7))
            * jnp.int32(NEWP) for g in range(7)]

    def g_start(j, idst, buf, sem):
        joff = j * jnp.int32(112)
        for g in range(7):
            v = idxv[pl.ds(joff + jnp.int32(16 * g), 16)]
            idst[pl.ds(jnp.int32(16 * g), 16)] = v + pats[g]
        pltpu.async_copy(pf.at[idst], buf, sem)

    def g_compute(buf, j):
        for d in range(16):
            for h in range(2):
                s = buf[7 * d, pl.ds(16 * h, 16)]
                for k in range(1, 7):
                    s = s + buf[7 * d + k, pl.ds(16 * h, 16)]
                acc[d, pl.ds(16 * h, 16)] = s
        pltpu.sync_copy(acc, out.at[pl.ds(obase + j * jnp.int32(16), 16)])

    g_start(jnp.int32(0), ia, ra, semA)

    def chunk(jj, carry):
        j0 = jj * jnp.int32(2)
        j1 = j0 + jnp.int32(1)
        g_start(j1, ib, rb, semB)
        pltpu.make_async_copy(pf.at[pl.ds(jnp.int32(0), 112)], ra, semA).wait()
        g_compute(ra, j0)

        @pl.when(jj < jnp.int32(GS_CHUNKS_PW // 2 - 1))
        def _():
            g_start(j0 + jnp.int32(2), ia, ra, semA)

        pltpu.make_async_copy(pf.at[pl.ds(jnp.int32(0), 112)], rb, semB).wait()
        g_compute(rb, j1)
        return carry
    lax.fori_loop(jnp.int32(0), jnp.int32(GS_CHUNKS_PW // 2), chunk,
                  jnp.int32(0))


def _mm_up_body(x_ref, w_ref, b_ref, o1_ref, o2_ref):
    x = x_ref[...]
    w = w_ref[...]
    b = b_ref[...]
    o = jnp.dot(x, w, preferred_element_type=F32) + b[0:1, :]
    o1_ref[...] = o[:, :224]
    o2_ref[...] = o[:, 224:]


def _mm_up(x1, wcat, bcat):
    bm = 2048
    grid = (pl.cdiv(RAW, bm),)
    return pl.pallas_call(
        _mm_up_body,
        grid=grid,
        in_specs=[pl.BlockSpec((bm, 64), lambda i: (i, jnp.int32(0))),
                  pl.BlockSpec((64, 336), lambda i: (jnp.int32(0), jnp.int32(0))),
                  pl.BlockSpec((8, 336), lambda i: (jnp.int32(0), jnp.int32(0)))],
        out_specs=[pl.BlockSpec((bm, 224), lambda i: (i, jnp.int32(0))),
                   pl.BlockSpec((bm, 112), lambda i: (i, jnp.int32(0)))],
        out_shape=[jax.ShapeDtypeStruct((RAW, 224), F32),
                   jax.ShapeDtypeStruct((RAW, 112), F32)],
    )(x1, wcat, bcat)


NBLK = NEWP // 2048


def _mm_p_body(xu_ref, x2_ref, a_ref, b_ref, o_ref):
    o_ref[...] = (jnp.dot(xu_ref[...], a_ref[0], preferred_element_type=F32)
                  + jnp.dot(x2_ref[...], b_ref[0], preferred_element_type=F32))


def _mm_p(xu4, x24, bda, bdb):
    return pl.pallas_call(
        _mm_p_body,
        grid=(NBLK, 7),
        in_specs=[pl.BlockSpec((512, 128), lambda i, k: (i, jnp.int32(0))),
                  pl.BlockSpec((512, 128), lambda i, k: (i, jnp.int32(0))),
                  pl.BlockSpec((1, 128, 128),
                               lambda i, k: (k, jnp.int32(0), jnp.int32(0))),
                  pl.BlockSpec((1, 128, 128),
                               lambda i, k: (k, jnp.int32(0), jnp.int32(0)))],
        out_specs=pl.BlockSpec((512, 128),
                               lambda i, k: (k * jnp.int32(NBLK) + i,
                                             jnp.int32(0))),
        out_shape=jax.ShapeDtypeStruct((7 * NEWP // 4, 128), F32),
    )(xu4, x24, bda, bdb)


BM_STATS = 2576
FULL_ROWS = NEW // 4


def _stats_body(y_ref, o_ref):
    i = pl.program_id(0)

    @pl.when(i == 0)
    def _():
        o_ref[...] = jnp.zeros_like(o_ref)

    blk = y_ref[...]
    rows = lax.broadcasted_iota(jnp.int32, blk.shape, 0) + i * BM_STATS
    lanes = lax.broadcasted_iota(jnp.int32, blk.shape, 1)
    m = jnp.logical_or(rows < FULL_ROWS,
                       jnp.logical_and(rows == FULL_ROWS, lanes < 64))
    v = jnp.where(m, blk, 0.0)
    s = jnp.sum(v, axis=0, keepdims=True)
    ss = jnp.sum(v * v, axis=0, keepdims=True)
    upd = jnp.pad(s, ((0, 7), (0, 0))) + jnp.pad(ss, ((1, 6), (0, 0)))
    o_ref[...] = o_ref[...] + upd


def _stats(y_pre_pad):
    y4 = y_pre_pad.reshape(V_PAD // 4, 128)
    grid = ((V_PAD // 4) // BM_STATS,)
    return pl.pallas_call(
        _stats_body,
        grid=grid,
        in_specs=[pl.BlockSpec((BM_STATS, 128), lambda i: (i, jnp.int32(0)))],
        out_specs=pl.BlockSpec((8, 128), lambda i: (jnp.int32(0), jnp.int32(0))),
        out_shape=jax.ShapeDtypeStruct((8, 128), F32),
    )(y4)


def _bn_coeffs(st_ref, gb_ref):
    st = st_ref[...]
    s4 = (st[0:1, 0:32] + st[0:1, 32:64] + st[0:1, 64:96] + st[0:1, 96:128])
    ss4 = (st[1:2, 0:32] + st[1:2, 32:64] + st[1:2, 64:96] + st[1:2, 96:128])
    mu = s4 * (1.0 / NEW)
    var = ss4 * (1.0 / NEW) - mu * mu
    inv = lax.rsqrt(var + 1e-5)
    gam = gb_ref[0:1, 0:32]
    bet = gb_ref[1:2, 0:32]
    scale = inv * gam
    shift = bet - mu * scale
    return scale, shift


def _conc4(v):
    return jnp.concatenate([v, v, v, v], axis=1)


def _norm_mm_body(y_ref, st_ref, gb_ref, w_ref, o_ref):
    scale, shift = _bn_coeffs(st_ref, gb_ref)
    t = y_ref[...] * _conc4(scale) + _conc4(shift)
    t = jnp.where(t >= 0, t, 0.2 * t)
    o_ref[...] = jnp.dot(t, w_ref[0], preferred_element_type=F32)


def _norm_mm(y4, st, gb, bd2):
    return pl.pallas_call(
        _norm_mm_body,
        grid=(NBLK, 7),
        in_specs=[pl.BlockSpec((512, 128), lambda i, k: (i, jnp.int32(0))),
                  pl.BlockSpec((8, 128),
                               lambda i, k: (jnp.int32(0), jnp.int32(0))),
                  pl.BlockSpec((8, 128),
                               lambda i, k: (jnp.int32(0), jnp.int32(0))),
                  pl.BlockSpec((1, 128, 128),
                               lambda i, k: (k, jnp.int32(0), jnp.int32(0)))],
        out_specs=pl.BlockSpec((512, 128),
                               lambda i, k: (k * jnp.int32(NBLK) + i,
                                             jnp.int32(0))),
        out_shape=jax.ShapeDtypeStruct((7 * NEWP // 4, 128), F32),
    )(y4, st, gb, bd2)


def _norm_body(z_ref, st_ref, gb_ref, lo_ref, hi_ref):
    scale, shift = _bn_coeffs(st_ref, gb_ref)
    t = z_ref[...] * scale + shift
    t = jnp.where(t >= 0, t, 0.2 * t)
    bits = lax.bitcast_convert_type(t, jnp.uint32)
    sign = bits & jnp.uint32(0x80000000)
    expo = lax.shift_right_logical(bits, jnp.uint32(23)) & jnp.uint32(0xFF)
    mant = bits & jnp.uint32(0x7FFFFF)
    normal = expo > jnp.uint32(0)
    hi = jnp.where(
        normal,
        sign | lax.shift_left(expo + jnp.uint32(896), jnp.uint32(20))
        | lax.shift_right_logical(mant, jnp.uint32(3)),
        sign)
    lo = jnp.where(normal, lax.shift_left(mant, jnp.uint32(29)), jnp.uint32(0))
    lo_ref[...] = lo
    hi_ref[...] = hi


def _norm(z_pre_pad, st, gb):
    bm = 2048
    grid = (pl.cdiv(NEW, bm),)
    return pl.pallas_call(
        _norm_body,
        grid=grid,
        in_specs=[pl.BlockSpec((bm, 32), lambda i: (i, jnp.int32(0))),
                  pl.BlockSpec((8, 128), lambda i: (jnp.int32(0), jnp.int32(0))),
                  pl.BlockSpec((8, 128), lambda i: (jnp.int32(0), jnp.int32(0)))],
        out_specs=[pl.BlockSpec((bm, 32), lambda i: (i, jnp.int32(0))),
                   pl.BlockSpec((bm, 32), lambda i: (i, jnp.int32(0)))],
        out_shape=[jax.ShapeDtypeStruct((NEW, 32), jnp.uint32),
                   jax.ShapeDtypeStruct((NEW, 32), jnp.uint32)],
    )(z_pre_pad, st, gb)


def kernel(x1, x2, neigh_orders, upconv_top_index, upconv_down_index,
           W_up, b_up, W_c1, b_c1, gamma1, beta1, W_c2, b_c2, gamma2, beta2):
    x1 = x1.astype(F32)
    x2 = x2.astype(F32)

    W_pa = W_up.reshape(64, 7, 16, 2).mean(-1).reshape(64, 112).astype(F32)
    b_pa = b_up.reshape(7, 16, 2).mean(-1).reshape(112).astype(F32)
    wcat = jnp.concatenate([W_up.astype(F32), W_pa], axis=1)
    bcat = jnp.zeros((8, 336), F32).at[0, :224].set(b_up.astype(F32))
    bcat = bcat.at[0, 224:].set(b_pa)
    w1r = W_c1.reshape(7, 64, OUT_CH).transpose(1, 0, 2).reshape(64, 224).astype(F32)
    eye4 = jnp.eye(4, dtype=F32)
    bda = jnp.stack([jnp.kron(eye4, w1r[:32, 32 * k:32 * k + 32])
                     for k in range(7)])
    bdb = jnp.stack([jnp.kron(eye4, w1r[32:, 32 * k:32 * k + 32])
                     for k in range(7)])
    w2r = W_c2.reshape(7, OUT_CH, OUT_CH).transpose(1, 0, 2).reshape(32, 224).astype(F32)
    bd2 = jnp.stack([jnp.kron(eye4, w2r[:, 32 * k:32 * k + 32])
                     for k in range(7)])
    gb1 = jnp.zeros((8, 128), F32).at[0, :32].set(gamma1.astype(F32))
    gb1 = gb1.at[1, :32].set(beta1.astype(F32))
    gb2 = jnp.zeros((8, 128), F32).at[0, :32].set(gamma2.astype(F32))
    gb2 = gb2.at[1, :32].set(beta2.astype(F32))

    ti = upconv_top_index.astype(I32)
    ti = jnp.concatenate([ti, jnp.zeros((TOP_PAD - RAW,), I32)])
    di = upconv_down_index.astype(I32)
    nidx = neigh_orders.astype(I32)
    nidx = jnp.concatenate([nidx, jnp.zeros((V_PAD * 7 - NEW * 7,), I32)])

    u1, u2 = _mm_up(x1, wcat, bcat)
    h32 = u1.reshape(TBL, 32)
    hpa = u2.reshape(TBL, 16)

    x_up = _upconv_gather(h32, hpa, ti, di)

    xu4 = x_up.reshape(NEWP // 4, 128)
    x24 = jnp.pad(x2, ((0, NEWP - NEW), (0, 0))).reshape(NEWP // 4, 128)
    p4 = _mm_p(xu4, x24, bda, bdb)
    y_pre = _gather_sum7(p4.reshape(7 * NEWP, 32), nidx)
    st1 = _stats(y_pre)
    y4 = y_pre.reshape(V_PAD // 4, 128)
    q4 = _norm_mm(y4, st1, gb1, bd2)

    z_pre = _gather_sum7(q4.reshape(7 * NEWP, 32), nidx)
    st2 = _stats(z_pre)
    lo, hi = _norm(z_pre, st2, gb2)
    pair = jnp.stack([lo, hi], axis=-1)
    return lax.bitcast_convert_type(pair, jnp.float64)


# speedup vs baseline: 1.3669x; 1.3669x over previous
import functools
import jax
import jax.numpy as jnp
from jax import lax
from jax.experimental import pallas as pl
from jax.experimental.pallas import tpu as pltpu
from jax.experimental.pallas import tpu_sc as plsc

F32 = jnp.float32
I32 = jnp.int32
RAW = 40962
NEW = 163842
OUT_CH = 32
TBL = RAW * 7

NW = 32
NEWP = 165888

TOP_CHUNKS_PW = 12
TOP_PER_W = TOP_CHUNKS_PW * 128
TOP_PAD = NW * TOP_PER_W
DOWN_N = (NEW - RAW) * 2
DOWN_CHUNKS_PW = 60
DOWN_PER_W = DOWN_CHUNKS_PW * 128

GS_CHUNKS_PW = 322
DEST_PER_W = GS_CHUNKS_PW * 16
V_PAD = NW * DEST_PER_W
IDX_PER_W = DEST_PER_W * 7

_sc_mesh = plsc.VectorSubcoreMesh(
    core_axis_name="c", subcore_axis_name="s", num_cores=2, num_subcores=16)


@functools.partial(
    pl.kernel,
    out_type=jax.ShapeDtypeStruct((NEWP, 32), F32),
    mesh=_sc_mesh,
    compiler_params=pltpu.CompilerParams(use_tc_tiling_on_sc=False),
    scratch_types=[pltpu.VMEM((TOP_PER_W,), I32),
                   pltpu.VMEM((DOWN_PER_W,), I32),
                   pltpu.VMEM((128, 32), F32),
                   pltpu.VMEM((128, 32), F32),
                   pltpu.VMEM((128, 16), F32),
                   pltpu.VMEM((128, 16), F32),
                   pltpu.VMEM((64, 32), F32),
                   pltpu.SemaphoreType.DMA,
                   pltpu.SemaphoreType.DMA],
)
def _upconv_gather(h32, hpa, ti, di, out,
                   tiv, div, ta, tb, da, db, bridge, semA, semB):
    wid = lax.axis_index("s") * jnp.int32(2) + lax.axis_index("c")

    tbase = wid * jnp.int32(TOP_PER_W)
    pltpu.sync_copy(ti.at[pl.ds(tbase, TOP_PER_W)], tiv)

    def t_start(j, buf, sem):
        pltpu.async_copy(h32.at[tiv.at[pl.ds(j * jnp.int32(128), 128)]],
                         buf, sem)

    def t_write(buf, j):
        off = tbase + j * jnp.int32(128)
        full = off + jnp.int32(128) <= jnp.int32(RAW)
        part = jnp.logical_and(jnp.logical_not(full), off < jnp.int32(RAW))

        @pl.when(full)
        def _():
            pltpu.sync_copy(buf, out.at[pl.ds(off, 128)])

        @pl.when(part)
        def _():
            pltpu.sync_copy(buf.at[pl.ds(jnp.int32(0), 2)],
                            out.at[pl.ds(off, 2)])

    t_start(jnp.int32(0), ta, semA)

    def t_body(jj, carry):
        j0 = jj * jnp.int32(2)
        j1 = j0 + jnp.int32(1)
        t_start(j1, tb, semB)
        pltpu.make_async_copy(h32.at[pl.ds(jnp.int32(0), 128)], ta, semA).wait()
        t_write(ta, j0)

        @pl.when(jj < jnp.int32(TOP_CHUNKS_PW // 2 - 1))
        def _():
            t_start(j0 + jnp.int32(2), ta, semA)

        pltpu.make_async_copy(h32.at[pl.ds(jnp.int32(0), 128)], tb, semB).wait()
        t_write(tb, j1)
        return carry
    lax.fori_loop(jnp.int32(0), jnp.int32(TOP_CHUNKS_PW // 2), t_body,
                  jnp.int32(0))

    dbase = wid * jnp.int32(DOWN_PER_W)
    obase = jnp.int32(RAW) + wid * jnp.int32(DOWN_PER_W // 2)
    pltpu.sync_copy(di.at[pl.ds(dbase, DOWN_PER_W)], div)

    def d_start(j, buf, sem):
        pltpu.async_copy(hpa.at[div.at[pl.ds(j * jnp.int32(128), 128)]],
                         buf, sem)

    def d_write(buf, j):
        for r in range(64):
            bridge[r, pl.ds(0, 16)] = buf[2 * r, pl.ds(0, 16)]
            bridge[r, pl.ds(16, 16)] = buf[2 * r + 1, pl.ds(0, 16)]
        pltpu.sync_copy(bridge, out.at[pl.ds(obase + j * jnp.int32(64), 64)])

    d_start(jnp.int32(0), da, semA)

    def d_body(jj, carry):
        j0 = jj * jnp.int32(2)
        j1 = j0 + jnp.int32(1)
        d_start(j1, db, semB)
        pltpu.make_async_copy(hpa.at[pl.ds(jnp.int32(0), 128)], da, semA).wait()
        d_write(da, j0)

        @pl.when(jj < jnp.int32(DOWN_CHUNKS_PW // 2 - 1))
        def _():
            d_start(j0 + jnp.int32(2), da, semA)

        pltpu.make_async_copy(hpa.at[pl.ds(jnp.int32(0), 128)], db, semB).wait()
        d_write(db, j1)
        return carry
    lax.fori_loop(jnp.int32(0), jnp.int32(DOWN_CHUNKS_PW // 2), d_body,
                  jnp.int32(0))


@functools.partial(
    pl.kernel,
    out_type=jax.ShapeDtypeStruct((V_PAD, 32), F32),
    mesh=_sc_mesh,
    compiler_params=pltpu.CompilerParams(use_tc_tiling_on_sc=False),
    scratch_types=[pltpu.VMEM((IDX_PER_W,), I32),
                   pltpu.VMEM((112,), I32),
                   pltpu.VMEM((112,), I32),
                   pltpu.VMEM((112, 32), F32),
                   pltpu.VMEM((112, 32), F32),
                   pltpu.VMEM((16, 32), F32),
                   pltpu.SemaphoreType.DMA,
                   pltpu.SemaphoreType.DMA],
)
def _gather_sum7(pf, nidx, out, idxv, ia, ib, ra, rb, acc, semA, semB):
    wid = lax.axis_index("s") * jnp.int32(2) + lax.axis_index("c")
    ibase = wid * jnp.int32(IDX_PER_W)
    obase = wid * jnp.int32(DEST_PER_W)
    pltpu.sync_copy(nidx.at[pl.ds(ibase, IDX_PER_W)], idxv)

    pats = [lax.rem(lax.iota(I32, 16) + jnp.int32(16 * g), jnp.int32(7))
            * jnp.int32(NEWP) for g in range(7)]

    def g_start(j, idst, buf, sem):
        joff = j * jnp.int32(112)
        for g in range(7):
            v = idxv[pl.ds(joff + jnp.int32(16 * g), 16)]
            idst[pl.ds(jnp.int32(16 * g), 16)] = v + pats[g]
        pltpu.async_copy(pf.at[idst], buf, sem)

    def g_compute(buf, j):
        for d in range(16):
            for h in range(2):
                s = buf[7 * d, pl.ds(16 * h, 16)]
                for k in range(1, 7):
                    s = s + buf[7 * d + k, pl.ds(16 * h, 16)]
                acc[d, pl.ds(16 * h, 16)] = s
        pltpu.sync_copy(acc, out.at[pl.ds(obase + j * jnp.int32(16), 16)])

    g_start(jnp.int32(0), ia, ra, semA)

    def chunk(jj, carry):
        j0 = jj * jnp.int32(2)
        j1 = j0 + jnp.int32(1)
        g_start(j1, ib, rb, semB)
        pltpu.make_async_copy(pf.at[pl.ds(jnp.int32(0), 112)], ra, semA).wait()
        g_compute(ra, j0)

        @pl.when(jj < jnp.int32(GS_CHUNKS_PW // 2 - 1))
        def _():
            g_start(j0 + jnp.int32(2), ia, ra, semA)

        pltpu.make_async_copy(pf.at[pl.ds(jnp.int32(0), 112)], rb, semB).wait()
        g_compute(rb, j1)
        return carry
    lax.fori_loop(jnp.int32(0), jnp.int32(GS_CHUNKS_PW // 2), chunk,
                  jnp.int32(0))


def _mm_up_body(x_ref, w_ref, b_ref, o1_ref, o2_ref):
    x = x_ref[...]
    w = w_ref[...]
    b = b_ref[...]
    o = jnp.dot(x, w, preferred_element_type=F32) + b[0:1, :]
    o1_ref[...] = o[:, :224]
    o2_ref[...] = o[:, 224:]


def _mm_up(x1, wcat, bcat):
    bm = 2048
    grid = (pl.cdiv(RAW, bm),)
    return pl.pallas_call(
        _mm_up_body,
        grid=grid,
        in_specs=[pl.BlockSpec((bm, 64), lambda i: (i, jnp.int32(0))),
                  pl.BlockSpec((64, 336), lambda i: (jnp.int32(0), jnp.int32(0))),
                  pl.BlockSpec((8, 336), lambda i: (jnp.int32(0), jnp.int32(0)))],
        out_specs=[pl.BlockSpec((bm, 224), lambda i: (i, jnp.int32(0))),
                   pl.BlockSpec((bm, 112), lambda i: (i, jnp.int32(0)))],
        out_shape=[jax.ShapeDtypeStruct((RAW, 224), F32),
                   jax.ShapeDtypeStruct((RAW, 112), F32)],
    )(x1, wcat, bcat)


NBLK = 8


def _mm_p_body(xu_ref, x2_ref, a_ref, b_ref, o_ref):
    o_ref[...] = (jnp.dot(xu_ref[...], a_ref[0], preferred_element_type=F32)
                  + jnp.dot(x2_ref[...], b_ref[0], preferred_element_type=F32))


def _mm_p(xu4, x24, bda, bdb):
    return pl.pallas_call(
        _mm_p_body,
        grid=(NBLK, 7),
        in_specs=[pl.BlockSpec((5184, 128), lambda i, k: (i, jnp.int32(0))),
                  pl.BlockSpec((5184, 128), lambda i, k: (i, jnp.int32(0))),
                  pl.BlockSpec((1, 128, 128),
                               lambda i, k: (k, jnp.int32(0), jnp.int32(0))),
                  pl.BlockSpec((1, 128, 128),
                               lambda i, k: (k, jnp.int32(0), jnp.int32(0)))],
        out_specs=pl.BlockSpec((5184, 128),
                               lambda i, k: (k * jnp.int32(NBLK) + i,
                                             jnp.int32(0))),
        out_shape=jax.ShapeDtypeStruct((7 * NEWP // 4, 128), F32),
    )(xu4, x24, bda, bdb)


BM_STATS = 2576
FULL_ROWS = NEW // 4


def _stats_body(y_ref, o_ref):
    i = pl.program_id(0)

    @pl.when(i == 0)
    def _():
        o_ref[...] = jnp.zeros_like(o_ref)

    blk = y_ref[...]
    rows = lax.broadcasted_iota(jnp.int32, blk.shape, 0) + i * BM_STATS
    lanes = lax.broadcasted_iota(jnp.int32, blk.shape, 1)
    m = jnp.logical_or(rows < FULL_ROWS,
                       jnp.logical_and(rows == FULL_ROWS, lanes < 64))
    v = jnp.where(m, blk, 0.0)
    s = jnp.sum(v, axis=0, keepdims=True)
    ss = jnp.sum(v * v, axis=0, keepdims=True)
    upd = jnp.pad(s, ((0, 7), (0, 0))) + jnp.pad(ss, ((1, 6), (0, 0)))
    o_ref[...] = o_ref[...] + upd


def _stats(y_pre_pad):
    y4 = y_pre_pad.reshape(V_PAD // 4, 128)
    grid = ((V_PAD // 4) // BM_STATS,)
    return pl.pallas_call(
        _stats_body,
        grid=grid,
        in_specs=[pl.BlockSpec((BM_STATS, 128), lambda i: (i, jnp.int32(0)))],
        out_specs=pl.BlockSpec((8, 128), lambda i: (jnp.int32(0), jnp.int32(0))),
        out_shape=jax.ShapeDtypeStruct((8, 128), F32),
    )(y4)


def _bn_coeffs(st_ref, gb_ref):
    st = st_ref[...]
    s4 = (st[0:1, 0:32] + st[0:1, 32:64] + st[0:1, 64:96] + st[0:1, 96:128])
    ss4 = (st[1:2, 0:32] + st[1:2, 32:64] + st[1:2, 64:96] + st[1:2, 96:128])
    mu = s4 * (1.0 / NEW)
    var = ss4 * (1.0 / NEW) - mu * mu
    inv = lax.rsqrt(var + 1e-5)
    gam = gb_ref[0:1, 0:32]
    bet = gb_ref[1:2, 0:32]
    scale = inv * gam
    shift = bet - mu * scale
    return scale, shift


def _conc4(v):
    return jnp.concatenate([v, v, v, v], axis=1)


def _norm_mm_body(y_ref, st_ref, gb_ref, w_ref, o_ref):
    scale, shift = _bn_coeffs(st_ref, gb_ref)
    t = y_ref[...] * _conc4(scale) + _conc4(shift)
    t = jnp.where(t >= 0, t, 0.2 * t)
    o_ref[...] = jnp.dot(t, w_ref[0], preferred_element_type=F32)


def _norm_mm(y4, st, gb, bd2):
    return pl.pallas_call(
        _norm_mm_body,
        grid=(NBLK, 7),
        in_specs=[pl.BlockSpec((5184, 128), lambda i, k: (i, jnp.int32(0))),
                  pl.BlockSpec((8, 128),
                               lambda i, k: (jnp.int32(0), jnp.int32(0))),
                  pl.BlockSpec((8, 128),
                               lambda i, k: (jnp.int32(0), jnp.int32(0))),
                  pl.BlockSpec((1, 128, 128),
                               lambda i, k: (k, jnp.int32(0), jnp.int32(0)))],
        out_specs=pl.BlockSpec((5184, 128),
                               lambda i, k: (k * jnp.int32(NBLK) + i,
                                             jnp.int32(0))),
        out_shape=jax.ShapeDtypeStruct((7 * NEWP // 4, 128), F32),
    )(y4, st, gb, bd2)


def _norm_body(z_ref, st_ref, gb_ref, lo_ref, hi_ref):
    scale, shift = _bn_coeffs(st_ref, gb_ref)
    t = z_ref[...] * scale + shift
    t = jnp.where(t >= 0, t, 0.2 * t)
    bits = lax.bitcast_convert_type(t, jnp.uint32)
    sign = bits & jnp.uint32(0x80000000)
    expo = lax.shift_right_logical(bits, jnp.uint32(23)) & jnp.uint32(0xFF)
    mant = bits & jnp.uint32(0x7FFFFF)
    normal = expo > jnp.uint32(0)
    hi = jnp.where(
        normal,
        sign | lax.shift_left(expo + jnp.uint32(896), jnp.uint32(20))
        | lax.shift_right_logical(mant, jnp.uint32(3)),
        sign)
    lo = jnp.where(normal, lax.shift_left(mant, jnp.uint32(29)), jnp.uint32(0))
    lo_ref[...] = lo
    hi_ref[...] = hi


def _norm(z_pre_pad, st, gb):
    bm = 2048
    grid = (pl.cdiv(NEW, bm),)
    return pl.pallas_call(
        _norm_body,
        grid=grid,
        in_specs=[pl.BlockSpec((bm, 32), lambda i: (i, jnp.int32(0))),
                  pl.BlockSpec((8, 128), lambda i: (jnp.int32(0), jnp.int32(0))),
                  pl.BlockSpec((8, 128), lambda i: (jnp.int32(0), jnp.int32(0)))],
        out_specs=[pl.BlockSpec((bm, 32), lambda i: (i, jnp.int32(0))),
                   pl.BlockSpec((bm, 32), lambda i: (i, jnp.int32(0)))],
        out_shape=[jax.ShapeDtypeStruct((NEW, 32), jnp.uint32),
                   jax.ShapeDtypeStruct((NEW, 32), jnp.uint32)],
    )(z_pre_pad, st, gb)


def kernel(x1, x2, neigh_orders, upconv_top_index, upconv_down_index,
           W_up, b_up, W_c1, b_c1, gamma1, beta1, W_c2, b_c2, gamma2, beta2):
    x1 = x1.astype(F32)
    x2 = x2.astype(F32)

    W_pa = W_up.reshape(64, 7, 16, 2).mean(-1).reshape(64, 112).astype(F32)
    b_pa = b_up.reshape(7, 16, 2).mean(-1).reshape(112).astype(F32)
    wcat = jnp.concatenate([W_up.astype(F32), W_pa], axis=1)
    bcat = jnp.zeros((8, 336), F32).at[0, :224].set(b_up.astype(F32))
    bcat = bcat.at[0, 224:].set(b_pa)
    w1r = W_c1.reshape(7, 64, OUT_CH).transpose(1, 0, 2).reshape(64, 224).astype(F32)
    eye4 = jnp.eye(4, dtype=F32)
    bda = jnp.stack([jnp.kron(eye4, w1r[:32, 32 * k:32 * k + 32])
                     for k in range(7)])
    bdb = jnp.stack([jnp.kron(eye4, w1r[32:, 32 * k:32 * k + 32])
                     for k in range(7)])
    w2r = W_c2.reshape(7, OUT_CH, OUT_CH).transpose(1, 0, 2).reshape(32, 224).astype(F32)
    bd2 = jnp.stack([jnp.kron(eye4, w2r[:, 32 * k:32 * k + 32])
                     for k in range(7)])
    gb1 = jnp.zeros((8, 128), F32).at[0, :32].set(gamma1.astype(F32))
    gb1 = gb1.at[1, :32].set(beta1.astype(F32))
    gb2 = jnp.zeros((8, 128), F32).at[0, :32].set(gamma2.astype(F32))
    gb2 = gb2.at[1, :32].set(beta2.astype(F32))

    ti = upconv_top_index.astype(I32)
    ti = jnp.concatenate([ti, jnp.zeros((TOP_PAD - RAW,), I32)])
    di = upconv_down_index.astype(I32)
    nidx = neigh_orders.astype(I32)
    nidx = jnp.concatenate([nidx, jnp.zeros((V_PAD * 7 - NEW * 7,), I32)])

    u1, u2 = _mm_up(x1, wcat, bcat)
    h32 = u1.reshape(TBL, 32)
    hpa = u2.reshape(TBL, 16)

    x_up = _upconv_gather(h32, hpa, ti, di)

    xu4 = x_up.reshape(NEWP // 4, 128)
    x24 = jnp.pad(x2, ((0, NEWP - NEW), (0, 0))).reshape(NEWP // 4, 128)
    p4 = _mm_p(xu4, x24, bda, bdb)
    y_pre = _gather_sum7(p4.reshape(7 * NEWP, 32), nidx)
    st1 = _stats(y_pre)
    y4 = y_pre.reshape(V_PAD // 4, 128)
    q4 = _norm_mm(y4, st1, gb1, bd2)

    z_pre = _gather_sum7(q4.reshape(7 * NEWP, 32), nidx)
    st2 = _stats(z_pre)
    lo, hi = _norm(z_pre, st2, gb2)
    pair = jnp.stack([lo, hi], axis=-1)
    return lax.bitcast_convert_type(pair, jnp.float64)
